# baseline (speedup 1.0000x reference)
import functools

import jax
import jax.numpy as jnp
from jax import lax
from jax.experimental import pallas as pl
from jax.experimental.pallas import tpu as pltpu
from jax.experimental.pallas import tpu_sc as plsc

_N = 10000
_D = 128
_HW = _D // 2
_E = 320000
_NC = 2
_NS = 16
_N_PAD = 10240
_ROWS_PER_TILE = _N_PAD // _NS
_EDGES_PER_TILE = _E // (_NC * _NS)
_CH = 80
_NCHUNK = _EDGES_PER_TILE // _CH
_NBUF = 5
_NGROUP = _NCHUNK // _NBUF
_CROWS = 32
_NCONV = _ROWS_PER_TILE // _CROWS


def _segment_sum_sc(hb, ei, zrows):
    mesh = plsc.VectorSubcoreMesh(core_axis_name="c", subcore_axis_name="s")

    @functools.partial(
        pl.kernel,
        out_type=[jax.ShapeDtypeStruct((_NC, _N_PAD, _HW), jnp.float32),
                  jax.ShapeDtypeStruct((_NC, _N_PAD, _HW), jnp.float32)],
        mesh=mesh,
        scratch_types=(
            [pltpu.VMEM_SHARED((_N_PAD, _D), jnp.bfloat16),
             pltpu.VMEM((_CROWS, _D), jnp.bfloat16),
             pltpu.VMEM((_CROWS, _HW), jnp.float32),
             pltpu.VMEM((_CROWS, _HW), jnp.float32)]
            + [pltpu.VMEM((_CH,), jnp.int32) for _ in range(2 * _NBUF)]
            + [pltpu.VMEM((_CH, _D), jnp.bfloat16) for _ in range(_NBUF)]
            + [pltpu.SemaphoreType.DMA for _ in range(3 * _NBUF)]
        ),
        compiler_params=pltpu.CompilerParams(use_tc_tiling_on_sc=False,
                                             needs_layout_passes=False),
    )
    def seg_kernel(h_hbm, ei_hbm, z_hbm, oute_hbm, outo_hbm, acc, cin, coute, couto, *scratch):
        sidx = scratch[0:_NBUF]
        didx = scratch[_NBUF:2 * _NBUF]
        rows = scratch[2 * _NBUF:3 * _NBUF]
        semi = scratch[3 * _NBUF:4 * _NBUF]
        semg = scratch[4 * _NBUF:5 * _NBUF]
        sema = scratch[5 * _NBUF:6 * _NBUF]
        c = lax.axis_index("c")
        s = lax.axis_index("s")
        wid = c * _NS + s
        pltpu.sync_copy(z_hbm, acc.at[pl.ds(s * _ROWS_PER_TILE, _ROWS_PER_TILE)])
        plsc.subcore_barrier()
        ebase = wid * _EDGES_PER_TILE

        @pl.loop(0, _NGROUP)
        def _group(g):
            cbase = ebase + g * (_NBUF * _CH)
            idx_cp = []
            for b in range(_NBUF):
                @pl.when(g > 0)
                def _(b=b):
                    pltpu.make_async_copy(rows[b], acc.at[didx[b]], sema[b]).wait()
                off = cbase + b * _CH
                idx_cp.append(
                    (pltpu.async_copy(ei_hbm.at[pl.ds(off, _CH)], sidx[b], semi[b]),
                     pltpu.async_copy(ei_hbm.at[pl.ds(_E + off, _CH)], didx[b], semi[b])))
            g_cp = []
            for b in range(_NBUF):
                idx_cp[b][0].wait()
                idx_cp[b][1].wait()
                g_cp.append(pltpu.async_copy(h_hbm.at[sidx[b]], rows[b], semg[b]))
            for b in range(_NBUF):
                g_cp[b].wait()
                pltpu.async_copy(rows[b], acc.at[didx[b]], sema[b], add=True)

        for b in range(_NBUF):
            pltpu.make_async_copy(rows[b], acc.at[didx[b]], sema[b]).wait()
        plsc.subcore_barrier()

        @pl.loop(0, _NCONV)
        def _conv(cc):
            rbase = s * _ROWS_PER_TILE + cc * _CROWS
            pltpu.sync_copy(acc.at[pl.ds(rbase, _CROWS)], cin)
            for r in range(_CROWS):
                for t in range(_D // 32):
                    v = cin[r, pl.ds(32 * t, 32)]
                    a, b2 = plsc.unpack(v, format=plsc.PackFormat.INTERLEAVED)
                    coute[r, pl.ds(16 * t, 16)] = a
                    couto[r, pl.ds(16 * t, 16)] = b2
            pltpu.sync_copy(coute, oute_hbm.at[c, pl.ds(rbase, _CROWS)])
            pltpu.sync_copy(couto, outo_hbm.at[c, pl.ds(rbase, _CROWS)])

    return seg_kernel(hb, ei, zrows)


_BLK = 1024
def _tc_layer(parts_e, parts_o, hb_prev, w_rel_e, w_rel_o, b_rel, w_root):
    def body(pe_ref, po_ref, h_ref, wre_ref, wro_ref, br_ref, wo_ref, o_ref):
        agg_e = pe_ref[0] + pe_ref[1]
        agg_o = po_ref[0] + po_ref[1]
        y = lax.dot_general(agg_e, wre_ref[...], (((1,), (1,)), ((), ())),
                            preferred_element_type=jnp.float32)
        y = y + lax.dot_general(agg_o, wro_ref[...], (((1,), (1,)), ((), ())),
                                preferred_element_type=jnp.float32)
        h = h_ref[...].astype(jnp.float32)
        y = y + lax.dot_general(h, wo_ref[...], (((1,), (1,)), ((), ())),
                                preferred_element_type=jnp.float32)
        y = jnp.maximum(y + br_ref[...], 0.0)
        o_ref[...] = y.astype(jnp.bfloat16)

    return pl.pallas_call(
        body,
        grid=(_N_PAD // _BLK,),
        in_specs=[
            pl.BlockSpec((_NC, _BLK, _HW), lambda i: (0, i, 0)),
            pl.BlockSpec((_NC, _BLK, _HW), lambda i: (0, i, 0)),
            pl.BlockSpec((_BLK, _D), lambda i: (i, 0)),
            pl.BlockSpec((_D, _HW), lambda i: (0, 0)),
            pl.BlockSpec((_D, _HW), lambda i: (0, 0)),
            pl.BlockSpec((1, _D), lambda i: (0, 0)),
            pl.BlockSpec((_D, _D), lambda i: (0, 0)),
        ],
        out_specs=pl.BlockSpec((_BLK, _D), lambda i: (i, 0)),
        out_shape=jax.ShapeDtypeStruct((_N_PAD, _D), jnp.bfloat16),
    )(parts_e, parts_o, hb_prev, w_rel_e, w_rel_o, b_rel.reshape(1, _D), w_root)


_FBLK = 1000


def _tc_final(parts_e, parts_o, hb_prev, w_rel_e, w_rel_o, b_rel, w_root, w_lin, b_lin):
    def body(pe_ref, po_ref, h_ref, wre_ref, wro_ref, br_ref, wo_ref, wl_ref, bl_ref, o_ref):
        y = lax.dot_general(pe_ref[0] + pe_ref[1], wre_ref[...], (((1,), (1,)), ((), ())),
                            preferred_element_type=jnp.float32)
        y = y + lax.dot_general(po_ref[0] + po_ref[1], wro_ref[...], (((1,), (1,)), ((), ())),
                                preferred_element_type=jnp.float32)
        h = h_ref[...].astype(jnp.float32)
        y = y + lax.dot_general(h, wo_ref[...], (((1,), (1,)), ((), ())),
                                preferred_element_type=jnp.float32)
        y = y + br_ref[...]
        z = lax.dot_general(y, wl_ref[...], (((1,), (1,)), ((), ())),
                            preferred_element_type=jnp.float32)
        o_ref[...] = z + bl_ref[...]

    return pl.pallas_call(
        body,
        grid=(_N // _FBLK,),
        in_specs=[
            pl.BlockSpec((_NC, _FBLK, _HW), lambda i: (0, i, 0)),
            pl.BlockSpec((_NC, _FBLK, _HW), lambda i: (0, i, 0)),
            pl.BlockSpec((_FBLK, _D), lambda i: (i, 0)),
            pl.BlockSpec((_D, _HW), lambda i: (0, 0)),
            pl.BlockSpec((_D, _HW), lambda i: (0, 0)),
            pl.BlockSpec((1, _D), lambda i: (0, 0)),
            pl.BlockSpec((_D, _D), lambda i: (0, 0)),
            pl.BlockSpec((_D, _D), lambda i: (0, 0)),
            pl.BlockSpec((1, _D), lambda i: (0, 0)),
        ],
        out_specs=pl.BlockSpec((_FBLK, _D), lambda i: (i, 0)),
        out_shape=jax.ShapeDtypeStruct((_N, _D), jnp.float32),
    )(parts_e, parts_o, hb_prev, w_rel_e, w_rel_o, b_rel.reshape(1, _D), w_root,
      w_lin, b_lin.reshape(1, _D))


def kernel(x, edge_index, W_rel1, b_rel1, W_root1, W_rel2, b_rel2, W_root2,
           W_rel3, b_rel3, W_root3, W_lin, b_lin):
    ei = edge_index.reshape(2 * _E)
    zrows = jnp.zeros((_ROWS_PER_TILE, _D), jnp.bfloat16)
    xb = jnp.pad(x, ((0, _N_PAD - _N), (0, 0))).astype(jnp.bfloat16)
    wr1e, wr1o = W_rel1[:, 0::2], W_rel1[:, 1::2]
    wr2e, wr2o = W_rel2[:, 0::2], W_rel2[:, 1::2]
    wr3e, wr3o = W_rel3[:, 0::2], W_rel3[:, 1::2]

    p1e, p1o = _segment_sum_sc(xb, ei, zrows)
    h1b = _tc_layer(p1e, p1o, xb, wr1e, wr1o, b_rel1, W_root1)
    p2e, p2o = _segment_sum_sc(h1b, ei, zrows)
    h2b = _tc_layer(p2e, p2o, h1b, wr2e, wr2o, b_rel2, W_root2)
    p3e, p3o = _segment_sum_sc(h2b, ei, zrows)
    return _tc_final(p3e, p3o, h2b, wr3e, wr3o, b_rel3, W_root3, W_lin, b_lin)


# speedup vs baseline: 1.1787x; 1.1787x over previous
import functools

import jax
import jax.numpy as jnp
from jax import lax
from jax.experimental import pallas as pl
from jax.experimental.pallas import tpu as pltpu
from jax.experimental.pallas import tpu_sc as plsc

_N = 10000
_D = 128
_HW = _D // 2
_E = 320000
_NC = 2
_NS = 16
_N_PAD = 10240
_ROWS_PER_TILE = _N_PAD // _NS
_EDGES_PER_TILE = _E // (_NC * _NS)
_CH = 80
_NCHUNK = _EDGES_PER_TILE // _CH
_NBUF = 5
_NGROUP = _NCHUNK // _NBUF
_CROWS = 32
_NCONV = _ROWS_PER_TILE // _CROWS


def _segment_sum_sc(hb, ei, zrows):
    mesh = plsc.VectorSubcoreMesh(core_axis_name="c", subcore_axis_name="s")

    @functools.partial(
        pl.kernel,
        out_type=jax.ShapeDtypeStruct((_NC, _N_PAD, _D), jnp.float32),
        mesh=mesh,
        scratch_types=(
            [pltpu.VMEM_SHARED((_N_PAD, _D), jnp.bfloat16),
             pltpu.VMEM((_CROWS, _D), jnp.bfloat16),
             pltpu.VMEM((_CROWS, _D), jnp.float32)]
            + [pltpu.VMEM((_CH,), jnp.int32) for _ in range(2 * _NBUF)]
            + [pltpu.VMEM((_CH, _D), jnp.bfloat16) for _ in range(_NBUF)]
            + [pltpu.SemaphoreType.DMA for _ in range(3 * _NBUF)]
        ),
        compiler_params=pltpu.CompilerParams(use_tc_tiling_on_sc=False,
                                             needs_layout_passes=False),
    )
    def seg_kernel(h_hbm, ei_hbm, z_hbm, out_hbm, acc, cin, cout, *scratch):
        sidx = scratch[0:_NBUF]
        didx = scratch[_NBUF:2 * _NBUF]
        rows = scratch[2 * _NBUF:3 * _NBUF]
        semi = scratch[3 * _NBUF:4 * _NBUF]
        semg = scratch[4 * _NBUF:5 * _NBUF]
        sema = scratch[5 * _NBUF:6 * _NBUF]
        c = lax.axis_index("c")
        s = lax.axis_index("s")
        wid = c * _NS + s
        pltpu.sync_copy(z_hbm, acc.at[pl.ds(s * _ROWS_PER_TILE, _ROWS_PER_TILE)])
        plsc.subcore_barrier()
        ebase = wid * _EDGES_PER_TILE

        @pl.loop(0, _NGROUP)
        def _group(g):
            cbase = ebase + g * (_NBUF * _CH)
            idx_cp = []
            for b in range(_NBUF):
                @pl.when(g > 0)
                def _(b=b):
                    pltpu.make_async_copy(rows[b], acc.at[didx[b]], sema[b]).wait()
                off = cbase + b * _CH
                idx_cp.append(
                    (pltpu.async_copy(ei_hbm.at[pl.ds(off, _CH)], sidx[b], semi[b]),
                     pltpu.async_copy(ei_hbm.at[pl.ds(_E + off, _CH)], didx[b], semi[b])))
            g_cp = []
            for b in range(_NBUF):
                idx_cp[b][0].wait()
                idx_cp[b][1].wait()
                g_cp.append(pltpu.async_copy(h_hbm.at[sidx[b]], rows[b], semg[b]))
            for b in range(_NBUF):
                g_cp[b].wait()
                pltpu.async_copy(rows[b], acc.at[didx[b]], sema[b], add=True)

        for b in range(_NBUF):
            pltpu.make_async_copy(rows[b], acc.at[didx[b]], sema[b]).wait()
        plsc.subcore_barrier()

        @pl.loop(0, _NCONV)
        def _conv(cc):
            rbase = s * _ROWS_PER_TILE + cc * _CROWS
            pltpu.sync_copy(acc.at[pl.ds(rbase, _CROWS)], cin)
            for r in range(_CROWS):
                for t in range(_D // 32):
                    v = cin[r, pl.ds(32 * t, 32)]
                    a, b2 = plsc.unpack(v, format=plsc.PackFormat.INTERLEAVED)
                    cout[r, pl.ds(16 * t, 16)] = a
                    cout[r, pl.ds(_HW + 16 * t, 16)] = b2
            pltpu.sync_copy(cout, out_hbm.at[c, pl.ds(rbase, _CROWS)])

    return seg_kernel(hb, ei, zrows)


_BLK = 1024
def _tc_layer(parts, hb_prev, w_relp, b_rel, w_root):
    def body(p_ref, h_ref, wr_ref, br_ref, wo_ref, o_ref):
        agg = p_ref[0] + p_ref[1]
        y = lax.dot_general(agg, wr_ref[...], (((1,), (1,)), ((), ())),
                            preferred_element_type=jnp.float32)
        h = h_ref[...].astype(jnp.float32)
        y = y + lax.dot_general(h, wo_ref[...], (((1,), (1,)), ((), ())),
                                preferred_element_type=jnp.float32)
        y = jnp.maximum(y + br_ref[...], 0.0)
        o_ref[...] = y.astype(jnp.bfloat16)

    return pl.pallas_call(
        body,
        grid=(_N_PAD // _BLK,),
        in_specs=[
            pl.BlockSpec((_NC, _BLK, _D), lambda i: (0, i, 0)),
            pl.BlockSpec((_BLK, _D), lambda i: (i, 0)),
            pl.BlockSpec((_D, _D), lambda i: (0, 0)),
            pl.BlockSpec((1, _D), lambda i: (0, 0)),
            pl.BlockSpec((_D, _D), lambda i: (0, 0)),
        ],
        out_specs=pl.BlockSpec((_BLK, _D), lambda i: (i, 0)),
        out_shape=jax.ShapeDtypeStruct((_N_PAD, _D), jnp.bfloat16),
    )(parts, hb_prev, w_relp, b_rel.reshape(1, _D), w_root)


_FBLK = 1000


def _tc_final(parts, hb_prev, w_relp, b_rel, w_root, w_lin, b_lin):
    def body(p_ref, h_ref, wr_ref, br_ref, wo_ref, wl_ref, bl_ref, o_ref):
        y = lax.dot_general(p_ref[0] + p_ref[1], wr_ref[...], (((1,), (1,)), ((), ())),
                            preferred_element_type=jnp.float32)
        h = h_ref[...].astype(jnp.float32)
        y = y + lax.dot_general(h, wo_ref[...], (((1,), (1,)), ((), ())),
                                preferred_element_type=jnp.float32)
        y = y + br_ref[...]
        z = lax.dot_general(y, wl_ref[...], (((1,), (1,)), ((), ())),
                            preferred_element_type=jnp.float32)
        o_ref[...] = z + bl_ref[...]

    return pl.pallas_call(
        body,
        grid=(_N // _FBLK,),
        in_specs=[
            pl.BlockSpec((_NC, _FBLK, _D), lambda i: (0, i, 0)),
            pl.BlockSpec((_FBLK, _D), lambda i: (i, 0)),
            pl.BlockSpec((_D, _D), lambda i: (0, 0)),
            pl.BlockSpec((1, _D), lambda i: (0, 0)),
            pl.BlockSpec((_D, _D), lambda i: (0, 0)),
            pl.BlockSpec((_D, _D), lambda i: (0, 0)),
            pl.BlockSpec((1, _D), lambda i: (0, 0)),
        ],
        out_specs=pl.BlockSpec((_FBLK, _D), lambda i: (i, 0)),
        out_shape=jax.ShapeDtypeStruct((_N, _D), jnp.float32),
    )(parts, hb_prev, w_relp, b_rel.reshape(1, _D), w_root,
      w_lin, b_lin.reshape(1, _D))


def kernel(x, edge_index, W_rel1, b_rel1, W_root1, W_rel2, b_rel2, W_root2,
           W_rel3, b_rel3, W_root3, W_lin, b_lin):
    ei = edge_index.reshape(2 * _E)
    zrows = jnp.zeros((_ROWS_PER_TILE, _D), jnp.bfloat16)
    xb = jnp.pad(x, ((0, _N_PAD - _N), (0, 0))).astype(jnp.bfloat16)
    perm = jnp.array([32 * t + 2 * k + e
                      for e in (0, 1) for t in range(4) for k in range(16)],
                     dtype=jnp.int32)
    wr1p = W_rel1[:, perm]
    wr2p = W_rel2[:, perm]
    wr3p = W_rel3[:, perm]

    p1 = _segment_sum_sc(xb, ei, zrows)
    h1b = _tc_layer(p1, xb, wr1p, b_rel1, W_root1)
    p2 = _segment_sum_sc(h1b, ei, zrows)
    h2b = _tc_layer(p2, h1b, wr2p, b_rel2, W_root2)
    p3 = _segment_sum_sc(h2b, ei, zrows)
    return _tc_final(p3, h2b, wr3p, b_rel3, W_root3, W_lin, b_lin)
